# Initial kernel scaffold; baseline (speedup 1.0000x reference)
import jax
import jax.numpy as jnp
from jax.experimental import pallas as pl


def kernel(features, adj, W, a_src, a_dst):
    raise NotImplementedError("write your pallas kernel here")


# speedup vs baseline: 2.2833x; 2.2833x over previous
import functools

import jax
import jax.numpy as jnp
from jax.experimental import pallas as pl
from jax.experimental.pallas import tpu as pltpu


def _prepass_body(x_ref, w_ref, asrc_ref, adst_ref, wh_ref, s_ref, d_ref, dmax_ref):
    i = pl.program_id(0)
    wh = jnp.dot(x_ref[...], w_ref[...], preferred_element_type=jnp.float32)
    wh_ref[...] = wh
    s_ref[...] = jnp.dot(wh, asrc_ref[...], preferred_element_type=jnp.float32)
    d = jnp.dot(wh, adst_ref[...], preferred_element_type=jnp.float32)
    d_ref[...] = d
    bmax = jnp.max(d, axis=0, keepdims=True)

    @pl.when(i == 0)
    def _():
        dmax_ref[...] = bmax

    @pl.when(i > 0)
    def _():
        dmax_ref[...] = jnp.maximum(dmax_ref[...], bmax)


def _main_body(adj_ref, s_ref, dt_ref, wh_ref, dmax_ref, out_ref, sum_ref,
               *, n, h_heads, d_dim, bn, n_col_blocks):
    c = pl.program_id(1)

    col0 = c * bn
    col_ids = col0 + jax.lax.broadcasted_iota(jnp.int32, (1, bn), 1)
    adjm = jnp.where(col_ids < n, adj_ref[...], 0.0)

    for h in range(h_heads):
        sl = slice(h * d_dim, (h + 1) * d_dim)
        sh = s_ref[:, h:h + 1]
        dh = dt_ref[h:h + 1, :]
        mh = s_ref[:, h:h + 1] + dmax_ref[0, h]
        mh = jnp.maximum(mh, 0.2 * mh)
        e = sh + dh
        e = jnp.maximum(e, 0.2 * e)
        p = jnp.exp(e - mh) * adjm
        part = jnp.dot(p, wh_ref[:, sl], preferred_element_type=jnp.float32)
        rs = jnp.sum(p, axis=1, keepdims=True)

        @pl.when(c == 0)
        def _(part=part, rs=rs, sl=sl, h=h):
            out_ref[:, sl] = part
            sum_ref[:, h:h + 1] = rs

        @pl.when(c > 0)
        def _(part=part, rs=rs, sl=sl, h=h):
            out_ref[:, sl] += part
            sum_ref[:, h:h + 1] += rs

    @pl.when(c == n_col_blocks - 1)
    def _():
        for h in range(h_heads):
            sl = slice(h * d_dim, (h + 1) * d_dim)
            y = out_ref[:, sl] / sum_ref[:, h:h + 1]
            out_ref[:, sl] = jnp.where(y > 0, y, jnp.exp(y) - 1.0)


def kernel(features, adj, W, a_src, a_dst):
    n, f = features.shape
    h_heads, _, d_dim = W.shape
    hd = h_heads * d_dim

    bm1 = 1000
    r1 = n // bm1
    w_cat = jnp.transpose(W, (1, 0, 2)).reshape(f, hd)
    asrc = jnp.zeros((hd, h_heads), jnp.float32)
    adst = jnp.zeros((hd, h_heads), jnp.float32)
    for h in range(h_heads):
        asrc = asrc.at[h * d_dim:(h + 1) * d_dim, h].set(a_src[h])
        adst = adst.at[h * d_dim:(h + 1) * d_dim, h].set(a_dst[h])

    wh, s, d, dmax = pl.pallas_call(
        _prepass_body,
        grid=(r1,),
        in_specs=[
            pl.BlockSpec((bm1, f), lambda i: (i, 0)),
            pl.BlockSpec((f, hd), lambda i: (0, 0)),
            pl.BlockSpec((hd, h_heads), lambda i: (0, 0)),
            pl.BlockSpec((hd, h_heads), lambda i: (0, 0)),
        ],
        out_specs=[
            pl.BlockSpec((bm1, hd), lambda i: (i, 0)),
            pl.BlockSpec((bm1, h_heads), lambda i: (i, 0)),
            pl.BlockSpec((bm1, h_heads), lambda i: (i, 0)),
            pl.BlockSpec((1, h_heads), lambda i: (0, 0)),
        ],
        out_shape=[
            jax.ShapeDtypeStruct((n, hd), jnp.float32),
            jax.ShapeDtypeStruct((n, h_heads), jnp.float32),
            jax.ShapeDtypeStruct((n, h_heads), jnp.float32),
            jax.ShapeDtypeStruct((1, h_heads), jnp.float32),
        ],
    )(features, w_cat, asrc, adst)

    bm, bn = 1024, 2048
    rr = pl.cdiv(n, bm)
    cc = pl.cdiv(n, bn)
    npad = cc * bn

    dt_pad = jnp.zeros((8, npad), jnp.float32).at[:h_heads, :n].set(d.T)
    wh_pad = jnp.zeros((npad, hd), jnp.float32).at[:n].set(wh)

    body = functools.partial(_main_body, n=n, h_heads=h_heads, d_dim=d_dim,
                             bn=bn, n_col_blocks=cc)
    out = pl.pallas_call(
        body,
        grid=(rr, cc),
        in_specs=[
            pl.BlockSpec((bm, bn), lambda r, c: (r, c)),
            pl.BlockSpec((bm, h_heads), lambda r, c: (r, 0)),
            pl.BlockSpec((8, bn), lambda r, c: (0, c)),
            pl.BlockSpec((bn, hd), lambda r, c: (c, 0)),
            pl.BlockSpec((1, h_heads), lambda r, c: (0, 0)),
        ],
        out_specs=pl.BlockSpec((bm, hd), lambda r, c: (r, 0)),
        out_shape=jax.ShapeDtypeStruct((n, hd), jnp.float32),
        scratch_shapes=[pltpu.VMEM((bm, h_heads), jnp.float32)],
        compiler_params=pltpu.CompilerParams(
            dimension_semantics=("arbitrary", "arbitrary"),
        ),
    )(adj, s, dt_pad, wh_pad, dmax)
    return out
